# baseline (speedup 1.0000x reference)
import functools

import jax
import jax.numpy as jnp
from jax import lax
from jax.experimental import pallas as pl
from jax.experimental.pallas import tpu as pltpu
from jax.experimental.pallas import tpu_sc as plsc

BETA = 0.25
NE = 8192
DE = 256
NT = 8192

BM = 256
BN = 1024
NB = NT // BM

BL = 1024

NW = 32


def _vq_body(z2_ref, w2_ref, z_ref, w_ref, idx_ref, enc_ref, cnt_ref,
             prev_ref, acc_ref):
    zb = z_ref[...]
    z2 = z2_ref[...]
    nblk = NE // BN
    nsub = BN // 128

    i = pl.program_id(0)

    @pl.when(i == 0)
    def _():
        acc_ref[...] = jnp.zeros((1, NE), jnp.float32)

    parg = prev_ref[...]
    col = lax.broadcasted_iota(jnp.int32, (BM, NE), 1)
    enc = jnp.where(parg == col, 1.0, 0.0).astype(jnp.float32)
    enc_ref[...] = enc

    @pl.when(i > 0)
    def _():
        acc_ref[...] = acc_ref[...] + jnp.sum(enc, axis=0, keepdims=True)

    @pl.when(i == NB)
    def _():
        cnt_ref[...] = acc_ref[...]

    run_min = jnp.full((BM, 128), jnp.inf, jnp.float32)
    run_arg = jnp.zeros((BM, 128), jnp.int32)
    for j in range(nblk):
        wb = w_ref[pl.ds(j * BN, BN), :]
        m = lax.dot_general(zb, wb, (((1,), (1,)), ((), ())),
                            preferred_element_type=jnp.float32)
        for s in range(nsub):
            ms = m[:, s * 128:(s + 1) * 128]
            w2s = w2_ref[:, pl.ds(j * BN + s * 128, 128)]
            d = (z2 + w2s) - 2.0 * ms
            mask = d < run_min
            run_min = jnp.minimum(run_min, d)
            tile = jnp.full((BM, 128), j * nsub + s, jnp.int32)
            run_arg = jnp.where(mask, tile, run_arg)

    lane = lax.broadcasted_iota(jnp.int32, (BM, 128), 1)
    full_idx = run_arg * 128 + lane
    gmin = jnp.min(run_min, axis=1, keepdims=True)
    amin = jnp.min(jnp.where(run_min == gmin, full_idx, NE),
                   axis=1, keepdims=True)
    idx_ref[...] = amin
    prev_ref[...] = amin


def _vq_stage1(z_flat, W, z2, w2):
    last = NB - 1
    return pl.pallas_call(
        _vq_body,
        grid=(NB + 1,),
        in_specs=[
            pl.BlockSpec((BM, 1), lambda i: (jnp.minimum(i, last), 0)),
            pl.BlockSpec((1, NE), lambda i: (0, 0)),
            pl.BlockSpec((BM, DE), lambda i: (jnp.minimum(i, last), 0)),
            pl.BlockSpec((NE, DE), lambda i: (0, 0)),
        ],
        out_specs=[
            pl.BlockSpec((BM, 1), lambda i: (jnp.minimum(i, last), 0)),
            pl.BlockSpec((BM, NE), lambda i: (jnp.maximum(i - 1, 0), 0)),
            pl.BlockSpec((1, NE), lambda i: (0, 0)),
        ],
        out_shape=[
            jax.ShapeDtypeStruct((NT, 1), jnp.int32),
            jax.ShapeDtypeStruct((NT, NE), jnp.float32),
            jax.ShapeDtypeStruct((1, NE), jnp.float32),
        ],
        scratch_shapes=[pltpu.VMEM((BM, 1), jnp.int32),
                        pltpu.VMEM((1, NE), jnp.float32)],
    )(z2, w2, z_flat, W)


@functools.lru_cache(maxsize=None)
def _make_sc_gather():
    mesh = plsc.VectorSubcoreMesh(core_axis_name="c", subcore_axis_name="s")

    @functools.partial(
        pl.kernel,
        mesh=mesh,
        out_type=jax.ShapeDtypeStruct((NT, DE), jnp.float32),
        scratch_types=[
            pltpu.VMEM((128,), jnp.int32),
            pltpu.VMEM((128,), jnp.int32),
            pltpu.VMEM((128, DE), jnp.float32),
            pltpu.VMEM((128, DE), jnp.float32),
            pltpu.SemaphoreType.DMA,
            pltpu.SemaphoreType.DMA,
        ],
    )
    def _sc_gather(idx_hbm, w_hbm, q_hbm, idx0, idx1, rows0, rows1,
                   sem0, sem1):
        wid = lax.axis_index("s") * 2 + lax.axis_index("c")
        base = wid * (NT // NW)
        pltpu.sync_copy(idx_hbm.at[pl.ds(base, 128)], idx0)
        pltpu.sync_copy(idx_hbm.at[pl.ds(base + 128, 128)], idx1)
        c0 = pltpu.async_copy(w_hbm.at[idx0], rows0, sem0)
        c1 = pltpu.async_copy(w_hbm.at[idx1], rows1, sem1)
        c0.wait()
        pltpu.sync_copy(rows0, q_hbm.at[pl.ds(base, 128)])
        c1.wait()
        pltpu.sync_copy(rows1, q_hbm.at[pl.ds(base + 128, 128)])

    return _sc_gather


def _gather_rows(idx_flat, W):
    return _make_sc_gather()(idx_flat, W)


def _st_body(q_ref, z_ref, cnt_ref, st_ref, loss_ref, perp_ref, acc_ref):
    i = pl.program_id(0)

    @pl.when(i == 0)
    def _():
        acc_ref[...] = jnp.zeros((1, 1), jnp.float32)

    q = q_ref[...]
    zb = z_ref[...]
    diff = q - zb
    st_ref[...] = zb + diff
    dd = diff * diff
    rows = jnp.sum(dd, axis=1, keepdims=True)
    acc_ref[...] = acc_ref[...] + jnp.sum(rows, axis=0, keepdims=True)

    @pl.when(i == pl.num_programs(0) - 1)
    def _():
        s = acc_ref[...] * (1.0 / (NT * DE))
        loss_ref[...] = s + BETA * s
        p = cnt_ref[...] * (1.0 / NT)
        ent = p * jnp.log(p + 1e-10)
        perp_ref[...] = jnp.exp(-jnp.sum(ent, axis=1, keepdims=True))


def _vq_stage3(q, z_flat, counts):
    grid = (NT // BL,)
    return pl.pallas_call(
        _st_body,
        grid=grid,
        in_specs=[
            pl.BlockSpec((BL, DE), lambda i: (i, 0)),
            pl.BlockSpec((BL, DE), lambda i: (i, 0)),
            pl.BlockSpec((1, NE), lambda i: (0, 0)),
        ],
        out_specs=[
            pl.BlockSpec((BL, DE), lambda i: (i, 0)),
            pl.BlockSpec((1, 1), lambda i: (0, 0)),
            pl.BlockSpec((1, 1), lambda i: (0, 0)),
        ],
        out_shape=[
            jax.ShapeDtypeStruct((NT, DE), jnp.float32),
            jax.ShapeDtypeStruct((1, 1), jnp.float32),
            jax.ShapeDtypeStruct((1, 1), jnp.float32),
        ],
        scratch_shapes=[pltpu.VMEM((1, 1), jnp.float32)],
    )(q, z_flat, counts)


def kernel(z, W):
    z_p = jnp.transpose(z, (0, 2, 3, 1))
    z_flat = z_p.reshape(-1, DE)
    z2 = jnp.sum(z_flat ** 2, axis=1, keepdims=True)
    w2 = jnp.sum(W ** 2, axis=1).reshape(1, NE)

    idx2d, encodings, counts = _vq_stage1(z_flat, W, z2, w2)
    encoding_indices = idx2d.reshape(-1)

    q = _gather_rows(encoding_indices, W)

    st, loss2d, perp2d = _vq_stage3(q, z_flat, counts)

    loss = loss2d.reshape(())
    perplexity = perp2d.reshape(())
    quantized_out = jnp.transpose(st.reshape(z_p.shape), (0, 3, 1, 2))
    return (loss, quantized_out, perplexity, encodings, encoding_indices)
